# baseline (speedup 1.0000x reference)
import functools

import jax
import jax.numpy as jnp
from jax.experimental import pallas as pl
from jax.experimental.pallas import tpu as pltpu

_N = 2048
_BLK = 512
_NBLK = _N // _BLK


def _body(x_ref, gamma_ref, beta_ref, conv1_ref, conv2_ref, gcn_w_ref,
          gcn_b_ref, lin_w_ref, lin_b_ref, out_ref,
          gcnbf_ref, wsum_ref, colsum_ref, xw_head_ref, q_head_ref,
          q_prev_ref):
    b = pl.program_id(0)

    @pl.when(b == 0)
    def _weights():
        gcnbf_ref[:] = gcn_w_ref[:].astype(jnp.bfloat16)
        wsum_ref[:] = (lin_w_ref[0:1024, :]
                       + lin_w_ref[1024:2048, :]).astype(jnp.bfloat16)

    xb = x_ref[:]
    inv_d = jnp.float32(1.0 / xb.shape[1])
    mu = jnp.sum(xb, axis=1, keepdims=True) * inv_d
    var = jnp.sum(xb * xb, axis=1, keepdims=True) * inv_d - mu * mu
    xn = ((xb - mu) * jax.lax.rsqrt(var + 1e-5) * gamma_ref[:]
          + beta_ref[:]).astype(jnp.bfloat16)
    txw = jnp.dot(xn[0:8, :], gcnbf_ref[:],
                  preferred_element_type=jnp.float32)
    tu = jnp.maximum(txw + gcn_b_ref[:], 0.0).astype(jnp.bfloat16)
    tq = jnp.maximum(
        jnp.dot(tu, wsum_ref[:], preferred_element_type=jnp.float32)
        + lin_b_ref[:], 0.0)[0:2, :]

    @pl.when(b > 0)
    def _store_prev():
        out_ref[pl.ds((b - 1) * _BLK, _BLK), :] = jnp.concatenate(
            [q_prev_ref[2:_BLK, :], tq], axis=0)

    xw = jnp.dot(xn, gcnbf_ref[:],
                 preferred_element_type=jnp.float32)
    ub = jnp.maximum(xw + gcn_b_ref[:], 0.0).astype(jnp.bfloat16)
    qf = jnp.maximum(
        jnp.dot(ub, wsum_ref[:], preferred_element_type=jnp.float32)
        + lin_b_ref[:], 0.0)

    ones_row = jnp.full((8, _BLK), 1.0, dtype=jnp.bfloat16)
    cs_part = jnp.dot(ones_row, xn, preferred_element_type=jnp.float32)

    @pl.when(b == 0)
    def _init():
        colsum_ref[:] = cs_part[0:1, :]
        xw_head_ref[:] = xw[0:8, :]
        q_head_ref[:] = tq
    @pl.when(b > 0)
    def _accum():
        colsum_ref[:] = colsum_ref[:] + cs_part[0:1, :]

    q_prev_ref[:] = qf

    @pl.when(b == _NBLK - 1)
    def _tail():
        out_ref[pl.ds((_NBLK - 1) * _BLK, _BLK), :] = jnp.concatenate(
            [qf[2:_BLK, :], q_head_ref[0:2, :]], axis=0)

        s = jnp.dot(colsum_ref[:].astype(jnp.bfloat16), gcnbf_ref[:],
                    preferred_element_type=jnp.float32)
        xw0 = xw_head_ref[0:1, :]
        xw1 = xw_head_ref[1:2, :]
        xw2 = xw_head_ref[2:3, :]
        f1 = jax.nn.softmax(conv1_ref[:], axis=1)
        f2 = jax.nn.softmax(conv2_ref[:], axis=1)
        nm1 = jnp.float32(_N - 1)
        nm2 = jnp.float32(_N - 2)

        def channel_rows(c):
            f1l, f1c, f1s = f1[c, 0], f1[c, 1], f1[c, 2]
            f2l, f2c, f2s = f2[c, 0], f2[c, 1], f2[c, 2]
            g = (f1l + f1c) * (f2l + f2c)
            a_sl = f1s * f2l
            a_sc = f1s * f2c
            a_cs = f1c * f2s
            a_clcc = f1c * (f2l + f2c)
            num0 = g * xw2 + a_sl * (s - xw0 - xw1) + a_sc * (s - xw1)
            deg0 = g + nm2 * a_sl + nm1 * a_sc
            r0 = jnp.where(deg0 == 0.0, 0.0, num0 / deg0)
            numN = a_clcc * xw1 + a_cs * (s - xw0)
            degN = a_clcc + nm1 * a_cs
            rN = jnp.where(degN == 0.0, 0.0, numN / degN)
            o0 = jnp.maximum(r0 + gcn_b_ref[:], 0.0)
            oN = jnp.maximum(rN + gcn_b_ref[:], 0.0)
            return o0, oN

        o0_a, oN_a = channel_rows(0)
        o0_b, oN_b = channel_rows(1)
        ch0 = jnp.concatenate([o0_a, oN_a], axis=0).astype(jnp.bfloat16)
        ch1 = jnp.concatenate([o0_b, oN_b], axis=0).astype(jnp.bfloat16)
        sp = jnp.maximum(
            jnp.dot(ch0, lin_w_ref[0:1024, :].astype(jnp.bfloat16),
                    preferred_element_type=jnp.float32)
            + jnp.dot(ch1, lin_w_ref[1024:2048, :].astype(jnp.bfloat16),
                      preferred_element_type=jnp.float32)
            + lin_b_ref[:], 0.0)
        out_ref[0:1, :] = sp[0:1, :]
        out_ref[_N - 1:_N, :] = sp[1:2, :]


@functools.partial(jax.jit, static_argnames=())
def kernel(x, ln_gamma, ln_beta, conv1_w, conv2_w, gcn_w, gcn_b, lin_w, lin_b):
    d = x.shape[1]
    dout = lin_w.shape[1]
    gamma2 = ln_gamma.reshape(1, d)
    beta2 = ln_beta.reshape(1, d)
    gcn_b2 = gcn_b.reshape(1, -1)
    lin_b2 = lin_b.reshape(1, -1)

    const = lambda i, j: pl.BlockSpec((i, j), lambda b: (0, 0))
    out = pl.pallas_call(
        _body,
        grid=(_NBLK,),
        in_specs=[
            pl.BlockSpec((_BLK, d), lambda b: (b, 0)),
            const(1, d),
            const(1, d),
            const(2, 3),
            const(2, 3),
            const(d, dout),
            const(1, dout),
            const(lin_w.shape[0], dout),
            const(1, dout),
        ],
        out_specs=pl.BlockSpec((_N, dout), lambda b: (0, 0)),
        out_shape=jax.ShapeDtypeStruct((_N, dout), jnp.float32),
        scratch_shapes=[
            pltpu.VMEM((d, dout), jnp.bfloat16),
            pltpu.VMEM((d, dout), jnp.bfloat16),
            pltpu.VMEM((1, d), jnp.float32),
            pltpu.VMEM((8, dout), jnp.float32),
            pltpu.VMEM((2, dout), jnp.float32),
            pltpu.VMEM((_BLK, dout), jnp.float32),
        ],
        compiler_params=pltpu.CompilerParams(
            dimension_semantics=("arbitrary",)),
    )(x, gamma2, beta2, conv1_w, conv2_w, gcn_w, gcn_b2, lin_w, lin_b2)
    return out


# speedup vs baseline: 1.2616x; 1.2616x over previous
import functools

import jax
import jax.numpy as jnp
from jax.experimental import pallas as pl
from jax.experimental.pallas import tpu as pltpu

_N = 2048
_BLK = 512
_NBLK = _N // _BLK


def _body(x_ref, gamma_ref, beta_ref, conv1_ref, conv2_ref, gcn_w_ref,
          gcn_b_ref, lin_w_ref, lin_b_ref, out_ref,
          gcnbf_ref, wsum_ref, colsum_ref, xw_head_ref, u_head_ref,
          u_prev_ref, linw_ref, qstage_ref, spstage_ref,
          lw_sem, q_sem, sp_sem):
    b = pl.program_id(0)

    lw_copy = pltpu.make_async_copy(lin_w_ref, linw_ref, lw_sem)

    @pl.when(b == 0)
    def _start():
        lw_copy.start()
        gcnbf_ref[:] = gcn_w_ref[:].astype(jnp.bfloat16)

    @pl.when(b == 1)
    def _mk_wsum():
        lw_copy.wait()
        wsum_ref[:] = (linw_ref[0:1024, :]
                       + linw_ref[1024:2048, :]).astype(jnp.bfloat16)

    xb = x_ref[:]
    inv_d = jnp.float32(1.0 / xb.shape[1])
    mu = jnp.sum(xb, axis=1, keepdims=True) * inv_d
    var = jnp.sum(xb * xb, axis=1, keepdims=True) * inv_d - mu * mu
    xn = ((xb - mu) * jax.lax.rsqrt(var + 1e-5) * gamma_ref[:]
          + beta_ref[:]).astype(jnp.bfloat16)
    xw = jnp.dot(xn, gcnbf_ref[:],
                 preferred_element_type=jnp.float32)
    ub = jnp.maximum(xw + gcn_b_ref[:], 0.0).astype(jnp.bfloat16)

    ones_row = jnp.full((8, _BLK), 1.0, dtype=jnp.bfloat16)
    cs_part = jnp.dot(ones_row, xn, preferred_element_type=jnp.float32)

    @pl.when(b == 0)
    def _init():
        colsum_ref[:] = cs_part[0:1, :]
        xw_head_ref[:] = xw[0:8, :]
        u_head_ref[:] = ub[0:8, :]

    @pl.when(b > 0)
    def _accum():
        colsum_ref[:] = colsum_ref[:] + cs_part[0:1, :]

    def q_copy(i):
        return pltpu.make_async_copy(
            qstage_ref.at[i % 2], out_ref.at[pl.ds(i * _BLK, _BLK), :],
            q_sem.at[i])

    @pl.when(b > 0)
    def _gemm2_prev():
        ush = jnp.concatenate([u_prev_ref[2:_BLK, :], ub[0:2, :]], axis=0)
        q = jnp.maximum(
            jnp.dot(ush, wsum_ref[:], preferred_element_type=jnp.float32)
            + lin_b_ref[:], 0.0)

        @pl.when(b >= 3)
        def _reclaim():
            q_copy(b - 3).wait()

        qstage_ref[(b - 1) % 2] = q
        q_copy(b - 1).start()

    u_prev_ref[:] = ub

    @pl.when(b == _NBLK - 1)
    def _tail():
        ush = jnp.concatenate([ub[2:_BLK, :], u_head_ref[0:2, :]], axis=0)
        q = jnp.maximum(
            jnp.dot(ush, wsum_ref[:], preferred_element_type=jnp.float32)
            + lin_b_ref[:], 0.0)
        q_copy(_NBLK - 3).wait()
        qstage_ref[(_NBLK - 1) % 2] = q
        q_copy(_NBLK - 1).start()

        s = jnp.dot(colsum_ref[:].astype(jnp.bfloat16), gcnbf_ref[:],
                    preferred_element_type=jnp.float32)
        xw0 = xw_head_ref[0:1, :]
        xw1 = xw_head_ref[1:2, :]
        xw2 = xw_head_ref[2:3, :]
        f1 = jax.nn.softmax(conv1_ref[:], axis=1)
        f2 = jax.nn.softmax(conv2_ref[:], axis=1)
        nm1 = jnp.float32(_N - 1)
        nm2 = jnp.float32(_N - 2)

        def channel_rows(c):
            f1l, f1c, f1s = f1[c, 0], f1[c, 1], f1[c, 2]
            f2l, f2c, f2s = f2[c, 0], f2[c, 1], f2[c, 2]
            g = (f1l + f1c) * (f2l + f2c)
            a_sl = f1s * f2l
            a_sc = f1s * f2c
            a_cs = f1c * f2s
            a_clcc = f1c * (f2l + f2c)
            num0 = g * xw2 + a_sl * (s - xw0 - xw1) + a_sc * (s - xw1)
            deg0 = g + nm2 * a_sl + nm1 * a_sc
            r0 = jnp.where(deg0 == 0.0, 0.0, num0 / deg0)
            numN = a_clcc * xw1 + a_cs * (s - xw0)
            degN = a_clcc + nm1 * a_cs
            rN = jnp.where(degN == 0.0, 0.0, numN / degN)
            o0 = jnp.maximum(r0 + gcn_b_ref[:], 0.0)
            oN = jnp.maximum(rN + gcn_b_ref[:], 0.0)
            return o0, oN

        o0_a, oN_a = channel_rows(0)
        o0_b, oN_b = channel_rows(1)
        ch0 = jnp.concatenate([o0_a, oN_a], axis=0).astype(jnp.bfloat16)
        ch1 = jnp.concatenate([o0_b, oN_b], axis=0).astype(jnp.bfloat16)
        sp = jnp.maximum(
            jnp.dot(ch0, linw_ref[0:1024, :].astype(jnp.bfloat16),
                    preferred_element_type=jnp.float32)
            + jnp.dot(ch1, linw_ref[1024:2048, :].astype(jnp.bfloat16),
                      preferred_element_type=jnp.float32)
            + lin_b_ref[:], 0.0)
        spstage_ref[0:2, :] = sp

        q_copy(_NBLK - 2).wait()
        q_copy(_NBLK - 1).wait()
        sp0 = pltpu.make_async_copy(spstage_ref.at[0:1, :],
                                    out_ref.at[0:1, :], sp_sem.at[0])
        spN = pltpu.make_async_copy(spstage_ref.at[1:2, :],
                                    out_ref.at[_N - 1:_N, :], sp_sem.at[1])
        sp0.start()
        spN.start()
        sp0.wait()
        spN.wait()


@functools.partial(jax.jit, static_argnames=())
def kernel(x, ln_gamma, ln_beta, conv1_w, conv2_w, gcn_w, gcn_b, lin_w, lin_b):
    d = x.shape[1]
    dout = lin_w.shape[1]
    gamma2 = ln_gamma.reshape(1, d)
    beta2 = ln_beta.reshape(1, d)
    gcn_b2 = gcn_b.reshape(1, -1)
    lin_b2 = lin_b.reshape(1, -1)

    const = lambda i, j: pl.BlockSpec((i, j), lambda b: (0, 0))
    out = pl.pallas_call(
        _body,
        grid=(_NBLK,),
        in_specs=[
            pl.BlockSpec((_BLK, d), lambda b: (b, 0)),
            const(1, d),
            const(1, d),
            const(2, 3),
            const(2, 3),
            const(d, dout),
            const(1, dout),
            pl.BlockSpec(memory_space=pltpu.MemorySpace.HBM),
            const(1, dout),
        ],
        out_specs=pl.BlockSpec(memory_space=pltpu.MemorySpace.HBM),
        out_shape=jax.ShapeDtypeStruct((_N, dout), jnp.float32),
        scratch_shapes=[
            pltpu.VMEM((d, dout), jnp.bfloat16),
            pltpu.VMEM((d, dout), jnp.bfloat16),
            pltpu.VMEM((1, d), jnp.float32),
            pltpu.VMEM((8, dout), jnp.float32),
            pltpu.VMEM((8, dout), jnp.bfloat16),
            pltpu.VMEM((_BLK, dout), jnp.bfloat16),
            pltpu.VMEM((2 * d, dout), jnp.float32),
            pltpu.VMEM((2, _BLK, dout), jnp.float32),
            pltpu.VMEM((8, dout), jnp.float32),
            pltpu.SemaphoreType.DMA,
            pltpu.SemaphoreType.DMA((_NBLK,)),
            pltpu.SemaphoreType.DMA((2,)),
        ],
        compiler_params=pltpu.CompilerParams(
            dimension_semantics=("arbitrary",)),
    )(x, gamma2, beta2, conv1_w, conv2_w, gcn_w, gcn_b2, lin_w, lin_b2)
    return out
